# Initial kernel scaffold; baseline (speedup 1.0000x reference)
import jax
import jax.numpy as jnp
from jax.experimental import pallas as pl


def kernel(nodes, neigh_idx, features_table, weight):
    raise NotImplementedError("write your pallas kernel here")


# speedup vs baseline: 1.1947x; 1.1947x over previous
import functools

import jax
import jax.numpy as jnp
from jax import lax
from jax.experimental import pallas as pl
from jax.experimental.pallas import tpu as pltpu
from jax.experimental.pallas import tpu_sc as plsc

NC = 2
NS = 16
NW = NC * NS
NB = 128


def _sc_gather_kernel(Bpad, D, nnei, cpw):
    nl = 1 + nnei
    mesh = plsc.VectorSubcoreMesh(
        core_axis_name="c", subcore_axis_name="s", num_cores=NC, num_subcores=NS
    )

    @functools.partial(
        pl.kernel,
        mesh=mesh,
        out_type=[
            jax.ShapeDtypeStruct((Bpad, D), jnp.float32),
            jax.ShapeDtypeStruct((Bpad, D), jnp.float32),
        ],
        scratch_types=[
            pltpu.VMEM((nl * NB,), jnp.int32),
            pltpu.VMEM((NB,), jnp.int32),
            pltpu.VMEM((NB, D), jnp.float32),
            pltpu.VMEM((NB, D), jnp.float32),
            pltpu.VMEM_SHARED((NS * NB, D), jnp.float32),
            pltpu.SemaphoreType.DMA,
        ],
    )
    def sc_k(idx_hbm, ident_hbm, table_hbm, self_out, nsum_out,
             idx_v, ident_v, selfbuf, nbuf, acc_sh, sem):
        cid = lax.axis_index("c")
        sid = lax.axis_index("s")
        wid = sid * NC + cid
        pltpu.sync_copy(ident_hbm.at[pl.ds(sid * NB, NB)], ident_v)

        @pl.loop(0, cpw)
        def chunk(c):
            base = (wid * cpw + c) * NB
            ibase = (wid * cpw + c) * nl * NB
            pltpu.sync_copy(idx_hbm.at[pl.ds(ibase, nl * NB)], idx_v)
            pltpu.async_copy(table_hbm.at[idx_v.at[pl.ds(0, NB)]], selfbuf, sem).wait()
            pltpu.sync_copy(selfbuf, self_out.at[pl.ds(base, NB)])
            pltpu.async_copy(table_hbm.at[idx_v.at[pl.ds(NB, NB)]], nbuf, sem).wait()
            pltpu.sync_copy(nbuf, acc_sh.at[pl.ds(sid * NB, NB)])
            for j in range(2, nnei + 1):
                pltpu.async_copy(
                    table_hbm.at[idx_v.at[pl.ds(j * NB, NB)]], nbuf, sem
                ).wait()
                pltpu.sync_copy(nbuf, acc_sh.at[ident_v], add=True)
            pltpu.sync_copy(acc_sh.at[pl.ds(sid * NB, NB)], nsum_out.at[pl.ds(base, NB)])

    return sc_k


def _tc_matmul(self_rows, nsum_rows, weight, inv_n):
    Bpad, D = self_rows.shape
    E = weight.shape[0]
    bs = 2048
    grid = Bpad // bs

    def tc_k(s_ref, m_ref, w_ref, o_ref):
        ws = w_ref[:, :D]
        wn = w_ref[:, D:]
        dn = (((1,), (1,)), ((), ()))
        acc = lax.dot_general(ws, s_ref[...], dn, preferred_element_type=jnp.float32)
        acc += inv_n * lax.dot_general(wn, m_ref[...], dn, preferred_element_type=jnp.float32)
        o_ref[...] = jnp.maximum(acc, 0.0)

    return pl.pallas_call(
        tc_k,
        grid=(grid,),
        in_specs=[
            pl.BlockSpec((bs, D), lambda i: (i, 0)),
            pl.BlockSpec((bs, D), lambda i: (i, 0)),
            pl.BlockSpec((E, 2 * D), lambda i: (0, 0)),
        ],
        out_specs=pl.BlockSpec((E, bs), lambda i: (0, i)),
        out_shape=jax.ShapeDtypeStruct((E, Bpad), jnp.float32),
    )(self_rows, nsum_rows, weight)


def kernel(nodes, neigh_idx, features_table, weight):
    B = nodes.shape[0]
    nnei = neigh_idx.shape[1]
    N, D = features_table.shape

    blk = NW * NB
    cpw = -(-B // blk)
    Bpad = blk * cpw

    idx_all = jnp.concatenate([nodes[:, None], neigh_idx], axis=1)
    idx_all = jnp.pad(idx_all, ((0, Bpad - B), (0, 0)))
    idx_flat = (
        idx_all.reshape(NW, cpw, NB, 1 + nnei).transpose(0, 1, 3, 2).reshape(-1)
    )
    ident = jnp.arange(NS * NB, dtype=jnp.int32)

    self_rows, nsum_rows = _sc_gather_kernel(Bpad, D, nnei, cpw)(
        idx_flat, ident, features_table
    )
    out = _tc_matmul(self_rows, nsum_rows, weight, 1.0 / nnei)
    return out[:, :B]


# speedup vs baseline: 1.2162x; 1.0180x over previous
import functools

import jax
import jax.numpy as jnp
from jax import lax
from jax.experimental import pallas as pl
from jax.experimental.pallas import tpu as pltpu
from jax.experimental.pallas import tpu_sc as plsc

NC = 2
NS = 16
NW = NC * NS
NB = 128


def _sc_gather_kernel(Bpad, D, nnei, cpw):
    nl = 1 + nnei
    mesh = plsc.VectorSubcoreMesh(
        core_axis_name="c", subcore_axis_name="s", num_cores=NC, num_subcores=NS
    )

    @functools.partial(
        pl.kernel,
        mesh=mesh,
        out_type=[
            jax.ShapeDtypeStruct((Bpad, D), jnp.float32),
            jax.ShapeDtypeStruct((Bpad, D), jnp.float32),
        ],
        scratch_types=[
            pltpu.VMEM((nl * NB,), jnp.int32),
            pltpu.VMEM((NB,), jnp.int32),
            pltpu.VMEM((NB, D), jnp.float32),
            pltpu.VMEM((NB, D), jnp.float32),
            pltpu.VMEM((NB, D), jnp.float32),
            pltpu.VMEM_SHARED((NS * NB, D), jnp.float32),
            pltpu.SemaphoreType.DMA,
            pltpu.SemaphoreType.DMA,
            pltpu.SemaphoreType.DMA,
        ],
    )
    def sc_k(idx_hbm, ident_hbm, table_hbm, self_out, nsum_out,
             idx_v, ident_v, selfbuf, nbufa, nbufb, acc_sh, sem_s, sem_a, sem_b):
        cid = lax.axis_index("c")
        sid = lax.axis_index("s")
        wid = sid * NC + cid
        pltpu.sync_copy(ident_hbm.at[pl.ds(sid * NB, NB)], ident_v)
        nbuf = {1: nbufa, 0: nbufb}
        sem = {1: sem_a, 0: sem_b}

        @pl.loop(0, cpw)
        def chunk(c):
            base = (wid * cpw + c) * NB
            ibase = (wid * cpw + c) * nl * NB
            pltpu.sync_copy(idx_hbm.at[pl.ds(ibase, nl * NB)], idx_v)

            def gather(jj, buf, s):
                return pltpu.async_copy(
                    table_hbm.at[idx_v.at[pl.ds(jj * NB, NB)]], buf, s
                )

            cp_s = gather(0, selfbuf, sem_s)
            cps = {1: gather(1, nbufa, sem_a), 0: gather(2, nbufb, sem_b)}
            for jj in range(1, nnei + 1):
                p = jj % 2
                cps[p].wait()
                if jj == 1:
                    pltpu.sync_copy(nbuf[p], acc_sh.at[pl.ds(sid * NB, NB)])
                else:
                    pltpu.sync_copy(nbuf[p], acc_sh.at[ident_v], add=True)
                if jj + 2 <= nnei:
                    cps[p] = gather(jj + 2, nbuf[p], sem[p])
            cp_s.wait()
            pltpu.sync_copy(selfbuf, self_out.at[pl.ds(base, NB)])
            pltpu.sync_copy(acc_sh.at[pl.ds(sid * NB, NB)], nsum_out.at[pl.ds(base, NB)])

    return sc_k


def _tc_matmul(self_rows, nsum_rows, weight, inv_n):
    Bpad, D = self_rows.shape
    E = weight.shape[0]
    bs = 2048
    grid = Bpad // bs

    def tc_k(s_ref, m_ref, w_ref, o_ref):
        ws = w_ref[:, :D]
        wn = w_ref[:, D:]
        dn = (((1,), (1,)), ((), ()))
        acc = lax.dot_general(ws, s_ref[...], dn, preferred_element_type=jnp.float32)
        acc += inv_n * lax.dot_general(wn, m_ref[...], dn, preferred_element_type=jnp.float32)
        o_ref[...] = jnp.maximum(acc, 0.0)

    return pl.pallas_call(
        tc_k,
        grid=(grid,),
        in_specs=[
            pl.BlockSpec((bs, D), lambda i: (i, 0)),
            pl.BlockSpec((bs, D), lambda i: (i, 0)),
            pl.BlockSpec((E, 2 * D), lambda i: (0, 0)),
        ],
        out_specs=pl.BlockSpec((E, bs), lambda i: (0, i)),
        out_shape=jax.ShapeDtypeStruct((E, Bpad), jnp.float32),
    )(self_rows, nsum_rows, weight)


def kernel(nodes, neigh_idx, features_table, weight):
    B = nodes.shape[0]
    nnei = neigh_idx.shape[1]
    N, D = features_table.shape

    blk = NW * NB
    cpw = -(-B // blk)
    Bpad = blk * cpw

    idx_all = jnp.concatenate([nodes[:, None], neigh_idx], axis=1)
    idx_all = jnp.pad(idx_all, ((0, Bpad - B), (0, 0)))
    idx_flat = (
        idx_all.reshape(NW, cpw, NB, 1 + nnei).transpose(0, 1, 3, 2).reshape(-1)
    )
    ident = jnp.arange(NS * NB, dtype=jnp.int32)

    self_rows, nsum_rows = _sc_gather_kernel(Bpad, D, nnei, cpw)(
        idx_flat, ident, features_table
    )
    out = _tc_matmul(self_rows, nsum_rows, weight, 1.0 / nnei)
    return out[:, :B]


# speedup vs baseline: 1.8233x; 1.4992x over previous
import functools

import jax
import jax.numpy as jnp
from jax import lax
from jax.experimental import pallas as pl
from jax.experimental.pallas import tpu as pltpu
from jax.experimental.pallas import tpu_sc as plsc

NC = 2
NS = 16
NW = NC * NS
NB = 64


def _sc_gather_kernel(Bpad, D, nnei, cpw):
    nl = 1 + nnei
    mesh = plsc.VectorSubcoreMesh(
        core_axis_name="c", subcore_axis_name="s", num_cores=NC, num_subcores=NS
    )

    @functools.partial(
        pl.kernel,
        mesh=mesh,
        out_type=[
            jax.ShapeDtypeStruct((Bpad, D), jnp.float32),
            jax.ShapeDtypeStruct((Bpad, D), jnp.float32),
        ],
        scratch_types=[
            pltpu.VMEM((nl * NB,), jnp.int32),
            pltpu.VMEM((NB,), jnp.int32),
            pltpu.VMEM((nl, NB, D), jnp.float32),
            pltpu.VMEM_SHARED((NS * NB, D), jnp.float32),
            pltpu.SemaphoreType.DMA((nl,)),
            pltpu.SemaphoreType.DMA,
            pltpu.SemaphoreType.DMA,
        ],
    )
    def sc_k(idx_hbm, ident_hbm, table_hbm, self_out, nsum_out,
             idx_v, ident_v, rows_v, acc_sh, sem_g, sem_a, sem_w):
        cid = lax.axis_index("c")
        sid = lax.axis_index("s")
        wid = sid * NC + cid
        pltpu.sync_copy(ident_hbm.at[pl.ds(sid * NB, NB)], ident_v)

        @pl.loop(0, cpw)
        def chunk(c):
            base = (wid * cpw + c) * NB
            ibase = (wid * cpw + c) * nl * NB
            pltpu.sync_copy(idx_hbm.at[pl.ds(ibase, nl * NB)], idx_v)
            cps = [
                pltpu.async_copy(
                    table_hbm.at[idx_v.at[pl.ds(jj * NB, NB)]], rows_v.at[jj],
                    sem_g.at[jj]
                )
                for jj in range(nl)
            ]
            cps[1].wait()
            pltpu.sync_copy(rows_v.at[1], acc_sh.at[pl.ds(sid * NB, NB)])
            cps[0].wait()
            cp_self = pltpu.async_copy(rows_v.at[0], self_out.at[pl.ds(base, NB)], sem_w)
            adds = []
            for jj in range(2, nl):
                cps[jj].wait()
                adds.append(
                    pltpu.async_copy(rows_v.at[jj], acc_sh.at[ident_v], sem_a, add=True)
                )
            for cp in adds:
                cp.wait()
            cp_self.wait()
            pltpu.sync_copy(acc_sh.at[pl.ds(sid * NB, NB)], nsum_out.at[pl.ds(base, NB)])

    return sc_k


def _tc_matmul(self_rows, nsum_rows, weight, inv_n):
    Bpad, D = self_rows.shape
    E = weight.shape[0]
    bs = 2048
    grid = Bpad // bs

    def tc_k(s_ref, m_ref, w_ref, o_ref):
        ws = w_ref[:, :D]
        wn = w_ref[:, D:]
        dn = (((1,), (1,)), ((), ()))
        acc = lax.dot_general(ws, s_ref[...], dn, preferred_element_type=jnp.float32)
        acc += inv_n * lax.dot_general(wn, m_ref[...], dn, preferred_element_type=jnp.float32)
        o_ref[...] = jnp.maximum(acc, 0.0)

    return pl.pallas_call(
        tc_k,
        grid=(grid,),
        in_specs=[
            pl.BlockSpec((bs, D), lambda i: (i, 0)),
            pl.BlockSpec((bs, D), lambda i: (i, 0)),
            pl.BlockSpec((E, 2 * D), lambda i: (0, 0)),
        ],
        out_specs=pl.BlockSpec((E, bs), lambda i: (0, i)),
        out_shape=jax.ShapeDtypeStruct((E, Bpad), jnp.float32),
    )(self_rows, nsum_rows, weight)


def kernel(nodes, neigh_idx, features_table, weight):
    B = nodes.shape[0]
    nnei = neigh_idx.shape[1]
    N, D = features_table.shape

    blk = NW * NB
    cpw = -(-B // blk)
    Bpad = blk * cpw

    idx_all = jnp.concatenate([nodes[:, None], neigh_idx], axis=1)
    idx_all = jnp.pad(idx_all, ((0, Bpad - B), (0, 0)))
    idx_flat = (
        idx_all.reshape(NW, cpw, NB, 1 + nnei).transpose(0, 1, 3, 2).reshape(-1)
    )
    ident = jnp.arange(NS * NB, dtype=jnp.int32)

    self_rows, nsum_rows = _sc_gather_kernel(Bpad, D, nnei, cpw)(
        idx_flat, ident, features_table
    )
    out = _tc_matmul(self_rows, nsum_rows, weight, 1.0 / nnei)
    return out[:, :B]
